# Initial kernel scaffold; baseline (speedup 1.0000x reference)
import jax
import jax.numpy as jnp
from jax.experimental import pallas as pl


def kernel(X, edge_index1, edge_index2, edge_index3, edge_index4, edge_index5, batch, Wr0, root0, b0, Wr1, root1, b1, Wr2, root2, b2, W1, bl1, W2, bl2, W3, bl3):
    raise NotImplementedError("write your pallas kernel here")


# speedup vs baseline: 2.7077x; 2.7077x over previous
import functools

import jax
import jax.numpy as jnp
from jax import lax
from jax.experimental import pallas as pl
from jax.experimental.pallas import tpu as pltpu
from jax.experimental.pallas import tpu_sc as plsc

N = 10000
E = 64000
R = 5
FIN = 162
H = 128
G = 128
FB = FIN - H

NC = 2
NS = 16
C = 128
NCHUNK = E // C
ROW_STRIDE = 624
ROW_BLKS = 5

BN = 1000


def _make_sc_aggregate(nrel):
  mesh = plsc.VectorSubcoreMesh(core_axis_name="c", subcore_axis_name="s")

  @functools.partial(
      pl.kernel,
      out_type=jax.ShapeDtypeStruct((nrel, N, H), jnp.float32),
      mesh=mesh,
      scratch_types=[
          pltpu.VMEM_SHARED((N, H), jnp.float32),
          pltpu.VMEM((C, H), jnp.float32),
          pltpu.VMEM((C, H), jnp.float32),
          pltpu.VMEM((C,), jnp.int32),
          pltpu.VMEM((C,), jnp.int32),
          pltpu.SemaphoreType.DMA,
      ],
  )
  def agg(tbl_hbm, src_hbm, dst_hbm, out_hbm, acc, rows, zbuf, sidx, didx, sem):
    cid = lax.axis_index("c")
    sid = lax.axis_index("s")

    zv = jnp.zeros((16,), jnp.float32)

    def zrow(i, carry):
      for j in range(H // 16):
        zbuf[i, pl.ds(j * 16, 16)] = zv
      return carry

    lax.fori_loop(0, C, zrow, 0)

    base = sid * ROW_STRIDE

    nj = (NCHUNK // NS) + jnp.where(sid < (NCHUNK % NS), 1, 0)

    for k in range(nrel):
      mine = (k % NC) == cid

      @pl.when(mine)
      def _zero():
        for kk in range(ROW_BLKS):
          pltpu.sync_copy(zbuf, acc.at[pl.ds(base + kk * C, C)])

      plsc.subcore_barrier()

      @pl.when(mine)
      def _accumulate():
        def body(j, carry):
          chunk = j * NS + sid
          b = k * E + chunk * C
          pltpu.sync_copy(src_hbm.at[pl.ds(b, C)], sidx)
          pltpu.async_copy(tbl_hbm.at[sidx], rows, sem).wait()
          pltpu.sync_copy(dst_hbm.at[pl.ds(b, C)], didx)
          pltpu.sync_copy(rows, acc.at[didx], add=True)
          return carry

        lax.fori_loop(0, nj, body, 0)

      plsc.subcore_barrier()

      @pl.when(mine)
      def _copy_out():
        for kk in range(ROW_BLKS):
          pltpu.sync_copy(acc.at[pl.ds(base + kk * C, C)],
                          out_hbm.at[k, pl.ds(base + kk * C, C)])

      plsc.subcore_barrier()

  return agg


_sc_aggregate10 = _make_sc_aggregate(2 * R)
_sc_aggregate5 = _make_sc_aggregate(R)


def _layer0_body(x_ref, s_ref, root_ref, wra_ref, wrb_ref, b_ref,
                 h_ref, inv_ref):
  x = x_ref[...]
  s = s_ref[...]
  acc = jnp.dot(x, root_ref[...], preferred_element_type=jnp.float32, precision=lax.Precision.HIGHEST)
  acc = acc + b_ref[...]
  invs = []
  for r in range(R):
    cnt = s[R + r, :, FB:FB + 1]
    inv = 1.0 / jnp.maximum(cnt, 1.0)
    invs.append(inv)
    acc = acc + jnp.dot(s[r] * inv, wra_ref[r],
                        preferred_element_type=jnp.float32, precision=lax.Precision.HIGHEST)
    acc = acc + jnp.dot(s[R + r] * inv, wrb_ref[r],
                        preferred_element_type=jnp.float32, precision=lax.Precision.HIGHEST)
  h_ref[...] = jnp.maximum(acc, 0.0)
  inv_ref[...] = jnp.stack(invs)


def _tc_layer0(X, S0, root, WrA, WrB, b):
  return pl.pallas_call(
      _layer0_body,
      grid=(N // BN,),
      in_specs=[
          pl.BlockSpec((BN, FIN), lambda i: (i, 0)),
          pl.BlockSpec((2 * R, BN, H), lambda i: (0, i, 0)),
          pl.BlockSpec((FIN, H), lambda i: (0, 0)),
          pl.BlockSpec((R, H, H), lambda i: (0, 0, 0)),
          pl.BlockSpec((R, H, H), lambda i: (0, 0, 0)),
          pl.BlockSpec((1, H), lambda i: (0, 0)),
      ],
      out_specs=[
          pl.BlockSpec((BN, H), lambda i: (i, 0)),
          pl.BlockSpec((R, BN, 1), lambda i: (0, i, 0)),
      ],
      out_shape=[
          jax.ShapeDtypeStruct((N, H), jnp.float32),
          jax.ShapeDtypeStruct((R, N, 1), jnp.float32),
      ],
  )(X, S0, root, WrA, WrB, b)


def _layer_body(x_ref, s_ref, inv_ref, root_ref, wr_ref, b_ref, h_ref):
  x = x_ref[...]
  s = s_ref[...]
  inv = inv_ref[...]
  acc = jnp.dot(x, root_ref[...], preferred_element_type=jnp.float32, precision=lax.Precision.HIGHEST)
  acc = acc + b_ref[...]
  for r in range(R):
    acc = acc + jnp.dot(s[r] * inv[r], wr_ref[r],
                        preferred_element_type=jnp.float32, precision=lax.Precision.HIGHEST)
  h_ref[...] = jnp.maximum(acc, 0.0)


def _tc_layer(h, S, inv, root, Wr, b):
  return pl.pallas_call(
      _layer_body,
      grid=(N // BN,),
      in_specs=[
          pl.BlockSpec((BN, H), lambda i: (i, 0)),
          pl.BlockSpec((R, BN, H), lambda i: (0, i, 0)),
          pl.BlockSpec((R, BN, 1), lambda i: (0, i, 0)),
          pl.BlockSpec((H, H), lambda i: (0, 0)),
          pl.BlockSpec((R, H, H), lambda i: (0, 0, 0)),
          pl.BlockSpec((1, H), lambda i: (0, 0)),
      ],
      out_specs=pl.BlockSpec((BN, H), lambda i: (i, 0)),
      out_shape=jax.ShapeDtypeStruct((N, H), jnp.float32),
  )(h, S, inv, root, Wr, b)


def _readout_body(h_ref, b_ref, w1_ref, b1_ref, w2_ref, b2_ref, w3_ref, b3_ref,
                  out_ref, acc):
  i = pl.program_id(0)

  @pl.when(i == 0)
  def _init():
    acc[...] = jnp.zeros_like(acc)

  bt = b_ref[...].reshape(1, BN)
  seg = (lax.broadcasted_iota(jnp.int32, (G, BN), 0) == bt)
  acc[...] += jnp.dot(seg.astype(jnp.float32), h_ref[...],
                      preferred_element_type=jnp.float32, precision=lax.Precision.HIGHEST)

  @pl.when(i == (N // BN) - 1)
  def _head():
    z = jnp.dot(acc[...], w1_ref[...], preferred_element_type=jnp.float32, precision=lax.Precision.HIGHEST)
    z = jnp.maximum(z + b1_ref[...], 0.0)
    z = jnp.dot(z, w2_ref[...], preferred_element_type=jnp.float32, precision=lax.Precision.HIGHEST)
    z = jnp.maximum(z + b2_ref[...], 0.0)
    z = jnp.dot(z, w3_ref[...], preferred_element_type=jnp.float32, precision=lax.Precision.HIGHEST)
    out_ref[...] = z + b3_ref[...]


def _tc_readout(h, batch3d, W1, bl1, W2, bl2, W3p, bl3p):
  return pl.pallas_call(
      _readout_body,
      grid=(N // BN,),
      in_specs=[
          pl.BlockSpec((BN, H), lambda i: (i, 0)),
          pl.BlockSpec((1, 1, BN), lambda i: (i, 0, 0)),
          pl.BlockSpec((H, H), lambda i: (0, 0)),
          pl.BlockSpec((1, H), lambda i: (0, 0)),
          pl.BlockSpec((H, H), lambda i: (0, 0)),
          pl.BlockSpec((1, H), lambda i: (0, 0)),
          pl.BlockSpec((H, H), lambda i: (0, 0)),
          pl.BlockSpec((1, H), lambda i: (0, 0)),
      ],
      out_specs=pl.BlockSpec((G, H), lambda i: (0, 0)),
      out_shape=jax.ShapeDtypeStruct((G, H), jnp.float32),
      scratch_shapes=[pltpu.VMEM((G, H), jnp.float32)],
  )(h, batch3d, W1, bl1, W2, bl2, W3p, bl3p)


def kernel(X, edge_index1, edge_index2, edge_index3, edge_index4, edge_index5,
           batch, Wr0, root0, b0, Wr1, root1, b1, Wr2, root2, b2,
           W1, bl1, W2, bl2, W3, bl3):
  eis = [edge_index1, edge_index2, edge_index3, edge_index4, edge_index5]
  srcs = jnp.concatenate([ei[0] for ei in eis])
  dsts = jnp.concatenate([ei[1] for ei in eis])

  ones_pad = jnp.zeros((N, H - FB), jnp.float32).at[:, 0].set(1.0)
  Xb = jnp.concatenate([X[:, H:], ones_pad], axis=1)
  T0 = jnp.concatenate([X[:, :H], Xb], axis=0)
  srcs0 = jnp.concatenate([srcs, srcs + N])
  dsts0 = jnp.concatenate([dsts, dsts])

  WrA = Wr0[:, :H, :]
  WrB = jnp.pad(Wr0[:, H:, :], ((0, 0), (0, H - FB), (0, 0)))

  S0 = _sc_aggregate10(T0, srcs0, dsts0)
  h, inv = _tc_layer0(X, S0, root0, WrA, WrB, b0.reshape(1, H))

  S1 = _sc_aggregate5(h, srcs, dsts)
  h = _tc_layer(h, S1, inv, root1, Wr1, b1.reshape(1, H))

  S2 = _sc_aggregate5(h, srcs, dsts)
  h = _tc_layer(h, S2, inv, root2, Wr2, b2.reshape(1, H))

  batch3d = batch.reshape(N // BN, 1, BN)
  W3p = jnp.pad(W3, ((0, 0), (0, H - 1)))
  bl3p = jnp.pad(bl3.reshape(1, 1), ((0, 0), (0, H - 1)))
  out = _tc_readout(h, batch3d, W1, bl1.reshape(1, H), W2, bl2.reshape(1, H),
                    W3p, bl3p)
  return out[:, 0:1]


# speedup vs baseline: 2.7377x; 1.0111x over previous
import functools

import jax
import jax.numpy as jnp
from jax import lax
from jax.experimental import pallas as pl
from jax.experimental.pallas import tpu as pltpu
from jax.experimental.pallas import tpu_sc as plsc

N = 10000
E = 64000
R = 5
FIN = 162
H = 128
G = 128
FB = FIN - H

NC = 2
NS = 16
C = 128
NCHUNK = E // C
ROW_STRIDE = 624
ROW_BLKS = 5

BN = 1000


def _make_sc_aggregate(nrel):
  mesh = plsc.VectorSubcoreMesh(core_axis_name="c", subcore_axis_name="s")

  @functools.partial(
      pl.kernel,
      out_type=jax.ShapeDtypeStruct((nrel, N, H), jnp.float32),
      mesh=mesh,
      scratch_types=[
          pltpu.VMEM_SHARED((N, H), jnp.float32),
          pltpu.VMEM((C, H), jnp.float32),
          pltpu.VMEM((C, H), jnp.float32),
          pltpu.VMEM((C,), jnp.int32),
          pltpu.VMEM((C,), jnp.int32),
          pltpu.SemaphoreType.DMA,
      ],
  )
  def agg(tbl_hbm, src_hbm, dst_hbm, out_hbm, acc, rows, zbuf, sidx, didx, sem):
    cid = lax.axis_index("c")
    sid = lax.axis_index("s")

    zv = jnp.zeros((16,), jnp.float32)

    def zrow(i, carry):
      for j in range(H // 16):
        zbuf[i, pl.ds(j * 16, 16)] = zv
      return carry

    lax.fori_loop(0, C, zrow, 0)

    base = sid * ROW_STRIDE

    nj = (NCHUNK // NS) + jnp.where(sid < (NCHUNK % NS), 1, 0)

    for k in range(nrel):
      mine = (k % NC) == cid

      @pl.when(mine)
      def _zero():
        for kk in range(ROW_BLKS):
          pltpu.sync_copy(zbuf, acc.at[pl.ds(base + kk * C, C)])

      plsc.subcore_barrier()

      @pl.when(mine)
      def _accumulate():
        def body(j, carry):
          chunk = j * NS + sid
          b = k * E + chunk * C
          pltpu.sync_copy(src_hbm.at[pl.ds(b, C)], sidx)
          pltpu.async_copy(tbl_hbm.at[sidx], rows, sem).wait()
          pltpu.sync_copy(dst_hbm.at[pl.ds(b, C)], didx)
          pltpu.sync_copy(rows, acc.at[didx], add=True)
          return carry

        lax.fori_loop(0, nj, body, 0)

      plsc.subcore_barrier()

      @pl.when(mine)
      def _copy_out():
        for kk in range(ROW_BLKS):
          pltpu.sync_copy(acc.at[pl.ds(base + kk * C, C)],
                          out_hbm.at[k, pl.ds(base + kk * C, C)])

      plsc.subcore_barrier()

  return agg


_sc_aggregate10 = _make_sc_aggregate(2 * R)
_sc_aggregate5 = _make_sc_aggregate(R)


def _layer0_body(x_ref, s_ref, root_ref, wra_ref, wrb_ref, b_ref,
                 h_ref, cm_ref):
  x16 = x_ref[...].astype(jnp.bfloat16)
  s = s_ref[...]
  acc = jnp.dot(x16, root_ref[...].astype(jnp.bfloat16),
                preferred_element_type=jnp.float32)
  acc = acc + b_ref[...]
  cms = []
  for r in range(R):
    cnt = s[R + r, :, FB:FB + 1]
    cm = jnp.maximum(cnt, 1.0)
    cms.append(cm)
    wa = wra_ref[r].astype(jnp.bfloat16).astype(jnp.float32)
    wb = wrb_ref[r].astype(jnp.bfloat16).astype(jnp.float32)
    t = jnp.dot(s[r], wa, preferred_element_type=jnp.float32,
                precision=lax.Precision.HIGHEST)
    t = t + jnp.dot(s[R + r], wb, preferred_element_type=jnp.float32,
                    precision=lax.Precision.HIGHEST)
    acc = acc + t / cm
  h = jnp.maximum(acc, 0.0)
  h_ref[...] = h.astype(jnp.bfloat16).astype(jnp.float32)
  cm_ref[...] = jnp.stack(cms)


def _tc_layer0(X, S0, root, WrA, WrB, b):
  return pl.pallas_call(
      _layer0_body,
      grid=(N // BN,),
      in_specs=[
          pl.BlockSpec((BN, FIN), lambda i: (i, 0)),
          pl.BlockSpec((2 * R, BN, H), lambda i: (0, i, 0)),
          pl.BlockSpec((FIN, H), lambda i: (0, 0)),
          pl.BlockSpec((R, H, H), lambda i: (0, 0, 0)),
          pl.BlockSpec((R, H, H), lambda i: (0, 0, 0)),
          pl.BlockSpec((1, H), lambda i: (0, 0)),
      ],
      out_specs=[
          pl.BlockSpec((BN, H), lambda i: (i, 0)),
          pl.BlockSpec((R, BN, 1), lambda i: (0, i, 0)),
      ],
      out_shape=[
          jax.ShapeDtypeStruct((N, H), jnp.float32),
          jax.ShapeDtypeStruct((R, N, 1), jnp.float32),
      ],
  )(X, S0, root, WrA, WrB, b)


def _make_layer_body(round_out):
  def _layer_body(x_ref, s_ref, cm_ref, root_ref, wr_ref, b_ref, h_ref):
    x16 = x_ref[...].astype(jnp.bfloat16)
    s = s_ref[...]
    cm = cm_ref[...]
    acc = jnp.dot(x16, root_ref[...].astype(jnp.bfloat16),
                  preferred_element_type=jnp.float32)
    acc = acc + b_ref[...]
    for r in range(R):
      w = wr_ref[r].astype(jnp.bfloat16).astype(jnp.float32)
      acc = acc + jnp.dot(s[r], w, preferred_element_type=jnp.float32,
                          precision=lax.Precision.HIGHEST) / cm[r]
    h = jnp.maximum(acc, 0.0)
    if round_out:
      h = h.astype(jnp.bfloat16).astype(jnp.float32)
    h_ref[...] = h

  return _layer_body


def _tc_layer(h, S, cm, root, Wr, b, round_out):
  return pl.pallas_call(
      _make_layer_body(round_out),
      grid=(N // BN,),
      in_specs=[
          pl.BlockSpec((BN, H), lambda i: (i, 0)),
          pl.BlockSpec((R, BN, H), lambda i: (0, i, 0)),
          pl.BlockSpec((R, BN, 1), lambda i: (0, i, 0)),
          pl.BlockSpec((H, H), lambda i: (0, 0)),
          pl.BlockSpec((R, H, H), lambda i: (0, 0, 0)),
          pl.BlockSpec((1, H), lambda i: (0, 0)),
      ],
      out_specs=pl.BlockSpec((BN, H), lambda i: (i, 0)),
      out_shape=jax.ShapeDtypeStruct((N, H), jnp.float32),
  )(h, S, cm, root, Wr, b)


def _readout_body(h_ref, b_ref, w1_ref, b1_ref, w2_ref, b2_ref, w3_ref, b3_ref,
                  out_ref, acc):
  i = pl.program_id(0)

  @pl.when(i == 0)
  def _init():
    acc[...] = jnp.zeros_like(acc)

  bt = b_ref[...].reshape(1, BN)
  seg = (lax.broadcasted_iota(jnp.int32, (G, BN), 0) == bt)
  acc[...] += jnp.dot(seg.astype(jnp.float32), h_ref[...],
                      preferred_element_type=jnp.float32, precision=lax.Precision.HIGHEST)

  @pl.when(i == (N // BN) - 1)
  def _head():
    z = jnp.dot(acc[...].astype(jnp.bfloat16),
                w1_ref[...].astype(jnp.bfloat16),
                preferred_element_type=jnp.float32)
    z = jnp.maximum(z + b1_ref[...], 0.0)
    z = jnp.dot(z.astype(jnp.bfloat16), w2_ref[...].astype(jnp.bfloat16),
                preferred_element_type=jnp.float32)
    z = jnp.maximum(z + b2_ref[...], 0.0)
    z = jnp.dot(z.astype(jnp.bfloat16), w3_ref[...].astype(jnp.bfloat16),
                preferred_element_type=jnp.float32)
    out_ref[...] = z + b3_ref[...]


def _tc_readout(h, batch3d, W1, bl1, W2, bl2, W3p, bl3p):
  return pl.pallas_call(
      _readout_body,
      grid=(N // BN,),
      in_specs=[
          pl.BlockSpec((BN, H), lambda i: (i, 0)),
          pl.BlockSpec((1, 1, BN), lambda i: (i, 0, 0)),
          pl.BlockSpec((H, H), lambda i: (0, 0)),
          pl.BlockSpec((1, H), lambda i: (0, 0)),
          pl.BlockSpec((H, H), lambda i: (0, 0)),
          pl.BlockSpec((1, H), lambda i: (0, 0)),
          pl.BlockSpec((H, H), lambda i: (0, 0)),
          pl.BlockSpec((1, H), lambda i: (0, 0)),
      ],
      out_specs=pl.BlockSpec((G, H), lambda i: (0, 0)),
      out_shape=jax.ShapeDtypeStruct((G, H), jnp.float32),
      scratch_shapes=[pltpu.VMEM((G, H), jnp.float32)],
  )(h, batch3d, W1, bl1, W2, bl2, W3p, bl3p)


def kernel(X, edge_index1, edge_index2, edge_index3, edge_index4, edge_index5,
           batch, Wr0, root0, b0, Wr1, root1, b1, Wr2, root2, b2,
           W1, bl1, W2, bl2, W3, bl3):
  eis = [edge_index1, edge_index2, edge_index3, edge_index4, edge_index5]
  srcs = jnp.concatenate([ei[0] for ei in eis])
  dsts = jnp.concatenate([ei[1] for ei in eis])

  Xc = lax.optimization_barrier(X.astype(jnp.bfloat16)).astype(jnp.float32)
  ones_pad = jnp.zeros((N, H - FB), jnp.float32).at[:, 0].set(1.0)
  Xb = jnp.concatenate([Xc[:, H:], ones_pad], axis=1)
  T0 = jnp.concatenate([Xc[:, :H], Xb], axis=0)
  srcs0 = jnp.concatenate([srcs, srcs + N])
  dsts0 = jnp.concatenate([dsts, dsts])

  WrA = Wr0[:, :H, :]
  WrB = jnp.pad(Wr0[:, H:, :], ((0, 0), (0, H - FB), (0, 0)))

  S0 = _sc_aggregate10(T0, srcs0, dsts0)
  h, cm = _tc_layer0(X, S0, root0, WrA, WrB, b0.reshape(1, H))

  S1 = _sc_aggregate5(h, srcs, dsts)
  h = _tc_layer(h, S1, cm, root1, Wr1, b1.reshape(1, H), True)

  S2 = _sc_aggregate5(h, srcs, dsts)
  h = _tc_layer(h, S2, cm, root2, Wr2, b2.reshape(1, H), False)

  batch3d = batch.reshape(N // BN, 1, BN)
  W3p = jnp.pad(W3, ((0, 0), (0, H - 1)))
  bl3p = jnp.pad(bl3.reshape(1, 1), ((0, 0), (0, H - 1)))
  out = _tc_readout(h, batch3d, W1, bl1.reshape(1, H), W2, bl2.reshape(1, H),
                    W3p, bl3p)
  return out[:, 0:1]
